# Initial kernel scaffold; baseline (speedup 1.0000x reference)
import jax
import jax.numpy as jnp
from jax.experimental import pallas as pl


def kernel(x, vals, rows, cols):
    raise NotImplementedError("write your pallas kernel here")


# speedup vs baseline: 7.7597x; 7.7597x over previous
import functools

import jax
import jax.numpy as jnp
from jax import lax
from jax.experimental import pallas as pl
from jax.experimental.pallas import tpu as pltpu
from jax.experimental.pallas import tpu_sc as plsc

MP, MM, FF, BB = 8192, 16384, 64, 16
NNZ = 134217
NC, NS, LANES = 2, 16, 16
CH = 128
NNZP = ((NNZ + NS * CH - 1) // (NS * CH)) * (NS * CH)
NNZ_PER_TILE = NNZP // NS
NCH = NNZ_PER_TILE // CH
B_PER_CORE = BB // NC
RPT = MP // NS
FV = FF // LANES


def _sc_body(x_hbm, rows_hbm, cols_hbm, vals_hbm, out_hbm,
             rows_v, cols_v, cabs_v, vals_v, gbuf, zeros_v, acc_sh):
    cid = lax.axis_index("c")
    sid = lax.axis_index("s")

    pltpu.sync_copy(rows_hbm.at[sid], rows_v)
    pltpu.sync_copy(cols_hbm.at[sid], cols_v)
    pltpu.sync_copy(vals_hbm.at[sid], vals_v)

    def zloop(i, _):
        for f in range(FV):
            zeros_v[i, pl.ds(f * LANES, LANES)] = jnp.zeros((LANES,), jnp.float32)
        return 0
    lax.fori_loop(0, RPT, zloop, 0)

    def batch_body(bi, _):
        b = cid * B_PER_CORE + bi

        pltpu.sync_copy(zeros_v, acc_sh.at[pl.ds(sid * RPT, RPT)])

        base = b * MM

        def cloop(i, _):
            j = i // (CH // LANES)
            k = i % (CH // LANES)
            sl = pl.ds(k * LANES, LANES)
            cabs_v[j, sl] = cols_v[j, sl] + base
            return 0
        lax.fori_loop(0, NNZ_PER_TILE // LANES, cloop, 0)

        plsc.subcore_barrier()

        def chunk_body(j, _):
            pltpu.sync_copy(x_hbm.at[cabs_v.at[j]], gbuf)

            def scale_body(g, _):
                v16 = vals_v[pl.ds(j * CH + g * LANES, LANES)]
                for k in range(LANES):
                    bv = lax.gather(
                        v16, jnp.full((LANES, 1), k, jnp.int32),
                        lax.GatherDimensionNumbers(
                            offset_dims=(), collapsed_slice_dims=(0,),
                            start_index_map=(0,)),
                        (1,), mode=lax.GatherScatterMode.PROMISE_IN_BOUNDS)
                    i = g * LANES + k
                    for f in range(FV):
                        sl = pl.ds(f * LANES, LANES)
                        gbuf[i, sl] = gbuf[i, sl] * bv
                return 0
            lax.fori_loop(0, CH // LANES, scale_body, 0)

            pltpu.sync_copy(gbuf, acc_sh.at[rows_v.at[j]], add=True)
            return 0
        lax.fori_loop(0, NCH, chunk_body, 0)

        plsc.subcore_barrier()

        pltpu.sync_copy(acc_sh.at[pl.ds(sid * RPT, RPT)],
                        out_hbm.at[pl.ds(b * MP + sid * RPT, RPT)])

        plsc.subcore_barrier()
        return 0

    lax.fori_loop(0, B_PER_CORE, batch_body, 0)


def kernel(x, vals, rows, cols):
    x2d = x.reshape(BB * MM, FF)
    pad = NNZP - NNZ
    rows3 = jnp.pad(rows, (0, pad)).reshape(NS, NCH, CH)
    cols3 = jnp.pad(cols, (0, pad)).reshape(NS, NCH, CH)
    vals2 = jnp.pad(vals, (0, pad)).reshape(NS, NNZ_PER_TILE)

    mesh = plsc.VectorSubcoreMesh(
        core_axis_name="c", subcore_axis_name="s",
        num_cores=NC, num_subcores=NS)

    f = functools.partial(
        pl.kernel,
        out_type=jax.ShapeDtypeStruct((BB * MP, FF), jnp.float32),
        mesh=mesh,
        compiler_params=pltpu.CompilerParams(use_tc_tiling_on_sc=False),
        scratch_types=[
            pltpu.VMEM((NCH, CH), jnp.int32),
            pltpu.VMEM((NCH, CH), jnp.int32),
            pltpu.VMEM((NCH, CH), jnp.int32),
            pltpu.VMEM((NNZ_PER_TILE,), jnp.float32),
            pltpu.VMEM((CH, FF), jnp.float32),
            pltpu.VMEM((RPT, FF), jnp.float32),
            pltpu.VMEM_SHARED((MP, FF), jnp.float32),
        ],
    )(_sc_body)

    out2d = f(x2d, rows3, cols3, vals2)
    return out2d.reshape(BB, MP, FF)


# speedup vs baseline: 10.7705x; 1.3880x over previous
import functools

import jax
import jax.numpy as jnp
from jax import lax
from jax.experimental import pallas as pl
from jax.experimental.pallas import tpu as pltpu
from jax.experimental.pallas import tpu_sc as plsc

MP, MM, FF, BB = 8192, 16384, 64, 16
NNZ = 134217
NC, NS, LANES = 2, 16, 16
CH = 128
NNZP = ((NNZ + NS * CH - 1) // (NS * CH)) * (NS * CH)
NNZ_PER_TILE = NNZP // NS
NCH = NNZ_PER_TILE // CH
B_PER_CORE = BB // NC
RPT = MP // NS
FV = FF // LANES
NB = 3
NB2 = 3
ZR = 128


def _sc_body(x_hbm, rows_hbm, cols_hbm, vals_hbm, out_hbm,
             rows_v, cols_v, cabs_v, vals_v,
             gbuf, sbuf, zeros_v, acc_sh, gsem, ssem):
    cid = lax.axis_index("c")
    sid = lax.axis_index("s")

    pltpu.sync_copy(rows_hbm.at[sid], rows_v)
    pltpu.sync_copy(cols_hbm.at[sid], cols_v)
    pltpu.sync_copy(vals_hbm.at[sid], vals_v)

    def zloop(i, _):
        for f in range(FV):
            zeros_v[i, pl.ds(f * LANES, LANES)] = jnp.zeros((LANES,), jnp.float32)
        return 0
    lax.fori_loop(0, ZR, zloop, 0)

    for r in range(RPT // ZR):
        pltpu.sync_copy(zeros_v,
                        acc_sh.at[pl.ds(sid * RPT + r * ZR, ZR)])
    plsc.subcore_barrier()

    def batch_body(bi, _):
        b = cid * B_PER_CORE + bi

        cbase = b * MM

        def cloop(i, _):
            j = i // (CH // LANES)
            k = i % (CH // LANES)
            sl = pl.ds(k * LANES, LANES)
            cabs_v[j, sl] = cols_v[j, sl] + cbase
            return 0
        lax.fori_loop(0, NNZ_PER_TILE // LANES, cloop, 0)

        for t in range(NB):
            pltpu.async_copy(x_hbm.at[cabs_v.at[t]], gbuf.at[t], gsem.at[t])

        def chunk_body(j, _):
            jc = j % NB
            js = j % NB2

            pltpu.make_async_copy(
                x_hbm.at[cabs_v.at[j]], gbuf.at[jc], gsem.at[jc]).wait()

            @pl.when(j >= NB2)
            def _():
                pltpu.make_async_copy(
                    sbuf.at[js], acc_sh.at[rows_v.at[j]], ssem.at[js]).wait()

            def scale_body(g, _):
                v16 = vals_v[pl.ds(j * CH + g * LANES, LANES)]
                for k in range(LANES):
                    bv = lax.gather(
                        v16, jnp.full((LANES, 1), k, jnp.int32),
                        lax.GatherDimensionNumbers(
                            offset_dims=(), collapsed_slice_dims=(0,),
                            start_index_map=(0,)),
                        (1,), mode=lax.GatherScatterMode.PROMISE_IN_BOUNDS)
                    i = g * LANES + k
                    for f in range(FV):
                        sl = pl.ds(f * LANES, LANES)
                        sbuf[js, i, sl] = gbuf[jc, i, sl] * bv
                return 0
            lax.fori_loop(0, CH // LANES, scale_body, 0)

            pltpu.async_copy(sbuf.at[js], acc_sh.at[rows_v.at[j]],
                             ssem.at[js], add=True)

            @pl.when(j + NB < NCH)
            def _():
                pltpu.async_copy(x_hbm.at[cabs_v.at[j + NB]], gbuf.at[jc],
                                 gsem.at[jc])
            return 0
        lax.fori_loop(0, NCH, chunk_body, 0)

        for t in range(NB2):
            j2 = NCH - NB2 + t
            pltpu.make_async_copy(
                sbuf.at[j2 % NB2], acc_sh.at[rows_v.at[j2]],
                ssem.at[j2 % NB2]).wait()

        plsc.subcore_barrier()

        pltpu.sync_copy(acc_sh.at[pl.ds(sid * RPT, RPT)],
                        out_hbm.at[pl.ds(b * MP + sid * RPT, RPT)])

        @pl.when(bi + 1 < B_PER_CORE)
        def _():
            for r in range(RPT // ZR):
                pltpu.sync_copy(
                    zeros_v, acc_sh.at[pl.ds(sid * RPT + r * ZR, ZR)])

        plsc.subcore_barrier()
        return 0

    lax.fori_loop(0, B_PER_CORE, batch_body, 0)


def kernel(x, vals, rows, cols):
    x2d = x.reshape(BB * MM, FF)
    pad = NNZP - NNZ
    rows3 = jnp.pad(rows, (0, pad)).reshape(NS, NCH, CH)
    cols3 = jnp.pad(cols, (0, pad)).reshape(NS, NCH, CH)
    vals2 = jnp.pad(vals, (0, pad)).reshape(NS, NNZ_PER_TILE)

    mesh = plsc.VectorSubcoreMesh(
        core_axis_name="c", subcore_axis_name="s",
        num_cores=NC, num_subcores=NS)

    f = functools.partial(
        pl.kernel,
        out_type=jax.ShapeDtypeStruct((BB * MP, FF), jnp.float32),
        mesh=mesh,
        compiler_params=pltpu.CompilerParams(use_tc_tiling_on_sc=False),
        scratch_types=[
            pltpu.VMEM((NCH, CH), jnp.int32),
            pltpu.VMEM((NCH, CH), jnp.int32),
            pltpu.VMEM((NCH, CH), jnp.int32),
            pltpu.VMEM((NNZ_PER_TILE,), jnp.float32),
            pltpu.VMEM((NB, CH, FF), jnp.float32),
            pltpu.VMEM((NB2, CH, FF), jnp.float32),
            pltpu.VMEM((ZR, FF), jnp.float32),
            pltpu.VMEM_SHARED((MP, FF), jnp.float32),
            pltpu.SemaphoreType.DMA((NB,)),
            pltpu.SemaphoreType.DMA((NB2,)),
        ],
    )(_sc_body)

    out2d = f(x2d, rows3, cols3, vals2)
    return out2d.reshape(BB, MP, FF)


# speedup vs baseline: 21.3833x; 1.9853x over previous
import functools

import jax
import jax.numpy as jnp
from jax import lax
from jax.experimental import pallas as pl
from jax.experimental.pallas import tpu as pltpu
from jax.experimental.pallas import tpu_sc as plsc

MP, MM, FF, BB = 8192, 16384, 64, 16
NNZ = 134217
NC, NS, LANES = 2, 16, 16
CH = 128
NNZP = ((NNZ + NS * CH - 1) // (NS * CH)) * (NS * CH)
NNZ_PER_TILE = NNZP // NS
NCH = NNZ_PER_TILE // CH
B_PER_CORE = BB // NC
RPT = MP // NS
FV = FF // LANES
NB = 3
NB2 = 3
ZR = 128


def _sc_body(x_hbm, rows_hbm, cols_hbm, vals_hbm, out_hbm,
             rows_v, cols_v, cabs_v, vals_v,
             gbuf, sbuf, zeros_v, acc_sh, gsem, ssem):
    cid = lax.axis_index("c")
    sid = lax.axis_index("s")

    pltpu.sync_copy(rows_hbm.at[sid], rows_v)
    pltpu.sync_copy(cols_hbm.at[sid], cols_v)
    pltpu.sync_copy(vals_hbm.at[sid], vals_v)

    def zloop(i, _):
        for f in range(FV):
            zeros_v[i, pl.ds(f * LANES, LANES)] = jnp.zeros((LANES,), jnp.float32)
        return 0
    lax.fori_loop(0, ZR, zloop, 0)

    for r in range(RPT // ZR):
        pltpu.sync_copy(zeros_v,
                        acc_sh.at[pl.ds(sid * RPT + r * ZR, ZR)])
    plsc.subcore_barrier()

    def batch_body(bi, _):
        b = cid * B_PER_CORE + bi

        cbase = b * MM

        def cloop(i, _):
            j = i // (CH // LANES)
            k = i % (CH // LANES)
            sl = pl.ds(k * LANES, LANES)
            cabs_v[j, sl] = cols_v[j, sl] + cbase
            return 0
        lax.fori_loop(0, NNZ_PER_TILE // LANES, cloop, 0)

        for t in range(NB):
            pltpu.async_copy(x_hbm.at[cabs_v.at[t]], gbuf.at[t], gsem.at[t])

        def round_body(jr, _):
            for u in range(NB):
                j = jr * NB + u

                pltpu.make_async_copy(
                    x_hbm.at[cabs_v.at[j]], gbuf.at[u], gsem.at[u]).wait()

                @pl.when(j >= NB2)
                def _():
                    pltpu.make_async_copy(
                        sbuf.at[u], acc_sh.at[rows_v.at[j]],
                        ssem.at[u]).wait()

                def scale_body(g, _):
                    v16 = vals_v[pl.ds(j * CH + g * LANES, LANES)]
                    for k in range(LANES):
                        bv = lax.gather(
                            v16, jnp.full((LANES, 1), k, jnp.int32),
                            lax.GatherDimensionNumbers(
                                offset_dims=(), collapsed_slice_dims=(0,),
                                start_index_map=(0,)),
                            (1,),
                            mode=lax.GatherScatterMode.PROMISE_IN_BOUNDS)
                        i = g * LANES + k
                        for f in range(FV):
                            sl = pl.ds(f * LANES, LANES)
                            sbuf[u, i, sl] = gbuf[u, i, sl] * bv
                    return 0
                lax.fori_loop(0, CH // LANES, scale_body, 0)

                pltpu.async_copy(sbuf.at[u], acc_sh.at[rows_v.at[j]],
                                 ssem.at[u], add=True)

                @pl.when(j + NB < NCH)
                def _():
                    pltpu.async_copy(x_hbm.at[cabs_v.at[j + NB]],
                                     gbuf.at[u], gsem.at[u])
            return 0
        lax.fori_loop(0, NCH // NB, round_body, 0)

        for t in range(NB2):
            j2 = NCH - NB2 + t
            pltpu.make_async_copy(
                sbuf.at[j2 % NB2], acc_sh.at[rows_v.at[j2]],
                ssem.at[j2 % NB2]).wait()

        plsc.subcore_barrier()

        pltpu.sync_copy(acc_sh.at[pl.ds(sid * RPT, RPT)],
                        out_hbm.at[pl.ds(b * MP + sid * RPT, RPT)])

        @pl.when(bi + 1 < B_PER_CORE)
        def _():
            for r in range(RPT // ZR):
                pltpu.sync_copy(
                    zeros_v, acc_sh.at[pl.ds(sid * RPT + r * ZR, ZR)])

        plsc.subcore_barrier()
        return 0

    lax.fori_loop(0, B_PER_CORE, batch_body, 0)


def kernel(x, vals, rows, cols):
    x2d = x.reshape(BB * MM, FF)
    pad = NNZP - NNZ
    rows3 = jnp.pad(rows, (0, pad)).reshape(NS, NCH, CH)
    cols3 = jnp.pad(cols, (0, pad)).reshape(NS, NCH, CH)
    vals2 = jnp.pad(vals, (0, pad)).reshape(NS, NNZ_PER_TILE)

    mesh = plsc.VectorSubcoreMesh(
        core_axis_name="c", subcore_axis_name="s",
        num_cores=NC, num_subcores=NS)

    f = functools.partial(
        pl.kernel,
        out_type=jax.ShapeDtypeStruct((BB * MP, FF), jnp.float32),
        mesh=mesh,
        compiler_params=pltpu.CompilerParams(use_tc_tiling_on_sc=False),
        scratch_types=[
            pltpu.VMEM((NCH, CH), jnp.int32),
            pltpu.VMEM((NCH, CH), jnp.int32),
            pltpu.VMEM((NCH, CH), jnp.int32),
            pltpu.VMEM((NNZ_PER_TILE,), jnp.float32),
            pltpu.VMEM((NB, CH, FF), jnp.float32),
            pltpu.VMEM((NB2, CH, FF), jnp.float32),
            pltpu.VMEM((ZR, FF), jnp.float32),
            pltpu.VMEM_SHARED((MP, FF), jnp.float32),
            pltpu.SemaphoreType.DMA((NB,)),
            pltpu.SemaphoreType.DMA((NB2,)),
        ],
    )(_sc_body)

    out2d = f(x2d, rows3, cols3, vals2)
    return out2d.reshape(BB, MP, FF)
